# Initial kernel scaffold; baseline (speedup 1.0000x reference)
import jax
import jax.numpy as jnp
from jax.experimental import pallas as pl


def kernel(node, adj, batch_ptr, eps, W1, b1, g1, be1, W2, b2, g2, be2, W3, b3, gn, bn):
    raise NotImplementedError("write your pallas kernel here")


# speedup vs baseline: 4.5343x; 4.5343x over previous
import functools

import jax
import jax.numpy as jnp
from jax import lax
from jax.experimental import pallas as pl
from jax.experimental.pallas import tpu as pltpu
from jax.experimental.pallas import tpu_sc as plsc

D = 128
CHUNK = 128
NC = 2
NS = 16
NW = NC * NS


def _agg_sc(node, src_p, dst_p, n_pad, cpw):
    rpt = n_pad // NS
    nzc = rpt // CHUNK

    mesh = plsc.VectorSubcoreMesh(core_axis_name="c", subcore_axis_name="s")

    @functools.partial(
        pl.kernel,
        mesh=mesh,
        out_type=jax.ShapeDtypeStruct((NC * n_pad, D), jnp.float32),
        scratch_types=[
            pltpu.VMEM((cpw, CHUNK), jnp.int32),
            pltpu.VMEM((cpw, CHUNK), jnp.int32),
            pltpu.VMEM((CHUNK, D), jnp.float32),
            pltpu.VMEM_SHARED((n_pad, D), jnp.float32),
            pltpu.SemaphoreType.DMA,
        ],
    )
    def agg(node_hbm, src_hbm, dst_hbm, out_hbm, src_v, dst_v, rows_v, acc, sem):
        c = lax.axis_index("c")
        s = lax.axis_index("s")
        wid = s * NC + c

        zero16 = jnp.zeros((16,), jnp.float32)

        def zbody(k, carry):
            i = k // (D // 16)
            j = k % (D // 16)
            rows_v[i, pl.ds(j * 16, 16)] = zero16
            return carry

        lax.fori_loop(0, CHUNK * (D // 16), zbody, 0)
        for k in range(nzc):
            pltpu.sync_copy(rows_v, acc.at[pl.ds(s * rpt + k * CHUNK, CHUNK)])
        plsc.subcore_barrier()

        pltpu.sync_copy(src_hbm.at[wid], src_v)
        pltpu.sync_copy(dst_hbm.at[wid], dst_v)

        def body(j, carry):
            pltpu.async_copy(node_hbm.at[src_v.at[j]], rows_v, sem).wait()
            pltpu.sync_copy(rows_v, acc.at[dst_v.at[j]], add=True)
            return carry

        lax.fori_loop(0, cpw, body, 0)
        plsc.subcore_barrier()

        for k in range(nzc):
            row = s * rpt + k * CHUNK
            pltpu.sync_copy(acc.at[pl.ds(row, CHUNK)],
                            out_hbm.at[pl.ds(c * n_pad + row, CHUNK)])

    return agg(node, src_p, dst_p).reshape(NC, n_pad, D)


def _mlp_body(scale_ref, x_ref, p0_ref, p1_ref,
              w1_ref, b1_ref, g1_ref, be1_ref,
              w2_ref, b2_ref, g2_ref, be2_ref,
              w3_ref, b3_ref, gn_ref, bn_ref, out_ref):
    def ln_relu(h, g, b):
        mu = jnp.mean(h, axis=1, keepdims=True)
        xc = h - mu
        var = jnp.mean(xc * xc, axis=1, keepdims=True)
        return jnp.maximum(xc * lax.rsqrt(var + 1e-5) * g + b, 0.0)

    dn = (((1,), (1,)), ((), ()))
    h = scale_ref[0, 0] * x_ref[...] + p0_ref[...] + p1_ref[...]
    h = lax.dot_general(h, w1_ref[...], dn, preferred_element_type=jnp.float32)
    h = ln_relu(h + b1_ref[...], g1_ref[...], be1_ref[...])
    h = lax.dot_general(h, w2_ref[...], dn, preferred_element_type=jnp.float32)
    h = ln_relu(h + b2_ref[...], g2_ref[...], be2_ref[...])
    h = lax.dot_general(h, w3_ref[...], dn, preferred_element_type=jnp.float32)
    out_ref[...] = ln_relu(h + b3_ref[...], gn_ref[...], bn_ref[...])


def kernel(node, adj, batch_ptr, eps, W1, b1, g1, be1, W2, b2, g2, be2,
           W3, b3, gn, bn):
    n, d = node.shape
    e = adj.shape[1]
    assert d == D

    cpw = -(-e // (NW * CHUNK))
    ep = NW * cpw * CHUNK
    src = adj[0].astype(jnp.int32)
    dst = adj[1].astype(jnp.int32)
    src_p = jnp.concatenate([src, jnp.zeros((ep - e,), jnp.int32)]).reshape(NW, cpw, CHUNK)
    dst_p = jnp.concatenate([dst, jnp.full((ep - e,), n, jnp.int32)]).reshape(NW, cpw, CHUNK)

    n_pad = -(-(n + 1) // (NS * CHUNK)) * NS * CHUNK
    parts = _agg_sc(node, src_p, dst_p, n_pad, cpw)

    scale = (1.0 + eps).astype(jnp.float32).reshape(1, 1)

    br = 512
    nb = -(-n // br)
    full = lambda shp: pl.BlockSpec(shp, lambda i: (0, 0))
    row_blk = pl.BlockSpec((br, D), lambda i: (i, 0))
    vec = lambda: full((1, D))

    out = pl.pallas_call(
        _mlp_body,
        grid=(nb,),
        in_specs=[
            full((1, 1)),
            row_blk,
            row_blk, row_blk,
            full((D, D)), vec(), vec(), vec(),
            full((D, D)), vec(), vec(), vec(),
            full((D, D)), vec(), vec(), vec(),
        ],
        out_specs=row_blk,
        out_shape=jax.ShapeDtypeStruct((n, D), jnp.float32),
    )(
        scale, node, parts[0, :n], parts[1, :n],
        W1, b1.reshape(1, D), g1.reshape(1, D), be1.reshape(1, D),
        W2, b2.reshape(1, D), g2.reshape(1, D), be2.reshape(1, D),
        W3, b3.reshape(1, D), gn.reshape(1, D), bn.reshape(1, D),
    )
    return out


# speedup vs baseline: 5.7358x; 1.2650x over previous
import functools

import jax
import jax.numpy as jnp
from jax import lax
from jax.experimental import pallas as pl
from jax.experimental.pallas import tpu as pltpu
from jax.experimental.pallas import tpu_sc as plsc

D = 128
CHUNK = 128
NC = 2
NS = 16
NW = NC * NS
FAST_FRAC = 0.647


def _agg_sc(node, src_p, dst_p, n_pad, cpw_buf, n0, n1):
    rpt = n_pad // NS
    nzc = rpt // CHUNK

    mesh = plsc.VectorSubcoreMesh(core_axis_name="c", subcore_axis_name="s")

    @functools.partial(
        pl.kernel,
        mesh=mesh,
        out_type=jax.ShapeDtypeStruct((NC * n_pad, D), jnp.float32),
        scratch_types=[
            pltpu.VMEM((cpw_buf, CHUNK), jnp.int32),
            pltpu.VMEM((cpw_buf, CHUNK), jnp.int32),
            pltpu.VMEM((CHUNK, D), jnp.float32),
            pltpu.VMEM_SHARED((n_pad, D), jnp.float32),
            pltpu.SemaphoreType.DMA,
        ],
    )
    def agg(node_hbm, src_hbm, dst_hbm, out_hbm, src_v, dst_v, rows_v, acc, sem):
        c = lax.axis_index("c")
        s = lax.axis_index("s")
        wid = s * NC + c
        my_cpw = jnp.where(c == 0, n0, n1)

        zero16 = jnp.zeros((16,), jnp.float32)

        def zbody(k, carry):
            i = k // (D // 16)
            j = k % (D // 16)
            rows_v[i, pl.ds(j * 16, 16)] = zero16
            return carry

        lax.fori_loop(0, CHUNK * (D // 16), zbody, 0)
        for k in range(nzc):
            pltpu.sync_copy(rows_v, acc.at[pl.ds(s * rpt + k * CHUNK, CHUNK)])
        plsc.subcore_barrier()

        pltpu.sync_copy(src_hbm.at[wid], src_v)
        pltpu.sync_copy(dst_hbm.at[wid], dst_v)

        def body(j, carry):
            pltpu.async_copy(node_hbm.at[src_v.at[j]], rows_v, sem).wait()
            pltpu.sync_copy(rows_v, acc.at[dst_v.at[j]], add=True)
            return carry

        lax.fori_loop(0, my_cpw, body, 0)
        plsc.subcore_barrier()

        for k in range(nzc):
            row = s * rpt + k * CHUNK
            pltpu.sync_copy(acc.at[pl.ds(row, CHUNK)],
                            out_hbm.at[pl.ds(c * n_pad + row, CHUNK)])

    return agg(node, src_p, dst_p).reshape(NC, n_pad, D)


def _mlp_body(scale_ref, x_ref, p0_ref, p1_ref,
              w1_ref, b1_ref, g1_ref, be1_ref,
              w2_ref, b2_ref, g2_ref, be2_ref,
              w3_ref, b3_ref, gn_ref, bn_ref, out_ref):
    def ln_relu(h, g, b):
        mu = jnp.mean(h, axis=1, keepdims=True)
        xc = h - mu
        var = jnp.mean(xc * xc, axis=1, keepdims=True)
        return jnp.maximum(xc * lax.rsqrt(var + 1e-5) * g + b, 0.0)

    dn = (((1,), (1,)), ((), ()))
    h = scale_ref[0, 0] * x_ref[...] + p0_ref[...] + p1_ref[...]
    h = lax.dot_general(h, w1_ref[...], dn, preferred_element_type=jnp.float32)
    h = ln_relu(h + b1_ref[...], g1_ref[...], be1_ref[...])
    h = lax.dot_general(h, w2_ref[...], dn, preferred_element_type=jnp.float32)
    h = ln_relu(h + b2_ref[...], g2_ref[...], be2_ref[...])
    h = lax.dot_general(h, w3_ref[...], dn, preferred_element_type=jnp.float32)
    out_ref[...] = ln_relu(h + b3_ref[...], gn_ref[...], bn_ref[...])


def kernel(node, adj, batch_ptr, eps, W1, b1, g1, be1, W2, b2, g2, be2,
           W3, b3, gn, bn):
    n, d = node.shape
    e = adj.shape[1]
    assert d == D

    tch = -(-e // CHUNK)
    n0 = max(1, round(tch * FAST_FRAC / NS))
    n1 = max(1, -(-max(tch - NS * n0, 1) // NS))
    tch_pad = NS * (n0 + n1)
    ep = tch_pad * CHUNK
    cpw_buf = max(n0, n1)

    src = adj[0].astype(jnp.int32)
    dst = adj[1].astype(jnp.int32)
    src_c = jnp.concatenate([src, jnp.zeros((ep - e,), jnp.int32)]).reshape(tch_pad, CHUNK)
    dst_c = jnp.concatenate([dst, jnp.full((ep - e,), n, jnp.int32)]).reshape(tch_pad, CHUNK)

    import numpy as np
    gidx = np.zeros((NW, cpw_buf), np.int32)
    for s_ in range(NS):
        for c_ in range(NC):
            w = s_ * NC + c_
            ncw = n0 if c_ == 0 else n1
            off = (0 if c_ == 0 else NS * n0) + s_ * ncw
            r = np.minimum(np.arange(cpw_buf), ncw - 1)
            gidx[w] = off + r
    src_p = src_c[gidx]
    dst_p = dst_c[gidx]

    n_pad = -(-(n + 1) // (NS * CHUNK)) * NS * CHUNK
    parts = _agg_sc(node, src_p, dst_p, n_pad, cpw_buf, n0, n1)

    scale = (1.0 + eps).astype(jnp.float32).reshape(1, 1)

    br = 512
    nb = -(-n // br)
    full = lambda shp: pl.BlockSpec(shp, lambda i: (0, 0))
    row_blk = pl.BlockSpec((br, D), lambda i: (i, 0))
    vec = lambda: full((1, D))

    out = pl.pallas_call(
        _mlp_body,
        grid=(nb,),
        in_specs=[
            full((1, 1)),
            row_blk,
            row_blk, row_blk,
            full((D, D)), vec(), vec(), vec(),
            full((D, D)), vec(), vec(), vec(),
            full((D, D)), vec(), vec(), vec(),
        ],
        out_specs=row_blk,
        out_shape=jax.ShapeDtypeStruct((n, D), jnp.float32),
    )(
        scale, node, parts[0, :n], parts[1, :n],
        W1, b1.reshape(1, D), g1.reshape(1, D), be1.reshape(1, D),
        W2, b2.reshape(1, D), g2.reshape(1, D), be2.reshape(1, D),
        W3, b3.reshape(1, D), gn.reshape(1, D), bn.reshape(1, D),
    )
    return out


# speedup vs baseline: 6.5390x; 1.1400x over previous
import functools

import jax
import jax.numpy as jnp
from jax import lax
from jax.experimental import pallas as pl
from jax.experimental.pallas import tpu as pltpu
from jax.experimental.pallas import tpu_sc as plsc

D = 128
CHUNK = 128
NC = 2
NS = 16
NW = NC * NS
FAST_FRAC = 0.627


def _agg_sc(node, src_c, dst_c, n_pad, tch, n0, n1):
    rpt = n_pad // NS
    nzc = rpt // CHUNK
    cpw_buf = -(-(max(n0, n1) + 8) // 8) * 8
    ta = src_c.shape[0]

    mesh = plsc.VectorSubcoreMesh(core_axis_name="c", subcore_axis_name="s")

    @functools.partial(
        pl.kernel,
        mesh=mesh,
        out_type=jax.ShapeDtypeStruct((NC * n_pad, D), jnp.float32),
        scratch_types=[
            pltpu.VMEM((cpw_buf, CHUNK), jnp.int32),
            pltpu.VMEM((cpw_buf, CHUNK), jnp.int32),
            pltpu.VMEM((CHUNK, D), jnp.float32),
            pltpu.VMEM_SHARED((n_pad, D), jnp.float32),
            pltpu.SemaphoreType.DMA,
        ],
    )
    def agg(node_hbm, src_hbm, dst_hbm, out_hbm, src_v, dst_v, rows_v, acc, sem):
        c = lax.axis_index("c")
        s = lax.axis_index("s")
        ncw = jnp.where(c == 0, n0, n1)
        o_w = jnp.where(c == 0, 0, NS * n0) + s * ncw
        my_cpw = jnp.maximum(0, jnp.minimum(ncw, tch - o_w))
        start = jnp.maximum(0, jnp.minimum(o_w, ta - cpw_buf))
        start = pl.multiple_of((start // 8) * 8, 8)
        base = o_w - start

        zero16 = jnp.zeros((16,), jnp.float32)

        def zbody(k, carry):
            i = k // (D // 16)
            j = k % (D // 16)
            rows_v[i, pl.ds(j * 16, 16)] = zero16
            return carry

        lax.fori_loop(0, CHUNK * (D // 16), zbody, 0)
        for k in range(nzc):
            pltpu.sync_copy(rows_v, acc.at[pl.ds(s * rpt + k * CHUNK, CHUNK)])
        plsc.subcore_barrier()

        pltpu.sync_copy(src_hbm.at[pl.ds(start, cpw_buf)], src_v)
        pltpu.sync_copy(dst_hbm.at[pl.ds(start, cpw_buf)], dst_v)

        def body(j, carry):
            pltpu.async_copy(node_hbm.at[src_v.at[base + j]], rows_v, sem).wait()
            pltpu.sync_copy(rows_v, acc.at[dst_v.at[base + j]], add=True)
            return carry

        lax.fori_loop(0, my_cpw, body, 0)
        plsc.subcore_barrier()

        for k in range(nzc):
            row = s * rpt + k * CHUNK
            pltpu.sync_copy(acc.at[pl.ds(row, CHUNK)],
                            out_hbm.at[pl.ds(c * n_pad + row, CHUNK)])

    return agg(node, src_c, dst_c).reshape(NC, n_pad, D)


def _mlp_body(scale_ref, x_ref, p0_ref, p1_ref,
              w1_ref, b1_ref, g1_ref, be1_ref,
              w2_ref, b2_ref, g2_ref, be2_ref,
              w3_ref, b3_ref, gn_ref, bn_ref, out_ref):
    def ln_relu(h, g, b):
        mu = jnp.mean(h, axis=1, keepdims=True)
        xc = h - mu
        var = jnp.mean(xc * xc, axis=1, keepdims=True)
        return jnp.maximum(xc * lax.rsqrt(var + 1e-5) * g + b, 0.0)

    dn = (((1,), (1,)), ((), ()))
    h = scale_ref[0, 0] * x_ref[...] + p0_ref[...] + p1_ref[...]
    h = lax.dot_general(h, w1_ref[...], dn, preferred_element_type=jnp.float32)
    h = ln_relu(h + b1_ref[...], g1_ref[...], be1_ref[...])
    h = lax.dot_general(h, w2_ref[...], dn, preferred_element_type=jnp.float32)
    h = ln_relu(h + b2_ref[...], g2_ref[...], be2_ref[...])
    h = lax.dot_general(h, w3_ref[...], dn, preferred_element_type=jnp.float32)
    out_ref[...] = ln_relu(h + b3_ref[...], gn_ref[...], bn_ref[...])


def kernel(node, adj, batch_ptr, eps, W1, b1, g1, be1, W2, b2, g2, be2,
           W3, b3, gn, bn):
    n, d = node.shape
    e = adj.shape[1]
    assert d == D

    tch = -(-e // CHUNK)
    n0 = max(1, round(tch * FAST_FRAC / NS))
    n1 = max(1, -(-max(tch - NS * n0, 1) // NS))

    src = adj[0].astype(jnp.int32)
    dst = adj[1].astype(jnp.int32)
    ta = ((tch + 7) // 8) * 8 + 8
    pad = ta * CHUNK - e
    src_c = jnp.concatenate([src, jnp.zeros((pad,), jnp.int32)]).reshape(ta, CHUNK)
    dst_c = jnp.concatenate([dst, jnp.full((pad,), n, jnp.int32)]).reshape(ta, CHUNK)

    n_pad = -(-(n + 1) // (NS * CHUNK)) * NS * CHUNK
    parts = _agg_sc(node, src_c, dst_c, n_pad, tch, n0, n1)

    scale = (1.0 + eps).astype(jnp.float32).reshape(1, 1)

    br = 512
    nb = -(-n // br)
    full = lambda shp: pl.BlockSpec(shp, lambda i: (0, 0))
    row_blk = pl.BlockSpec((br, D), lambda i: (i, 0))
    vec = lambda: full((1, D))

    out = pl.pallas_call(
        _mlp_body,
        grid=(nb,),
        in_specs=[
            full((1, 1)),
            row_blk,
            row_blk, row_blk,
            full((D, D)), vec(), vec(), vec(),
            full((D, D)), vec(), vec(), vec(),
            full((D, D)), vec(), vec(), vec(),
        ],
        out_specs=row_blk,
        out_shape=jax.ShapeDtypeStruct((n, D), jnp.float32),
    )(
        scale, node, parts[0, :n], parts[1, :n],
        W1, b1.reshape(1, D), g1.reshape(1, D), be1.reshape(1, D),
        W2, b2.reshape(1, D), g2.reshape(1, D), be2.reshape(1, D),
        W3, b3.reshape(1, D), gn.reshape(1, D), bn.reshape(1, D),
    )
    return out


# speedup vs baseline: 6.9598x; 1.0643x over previous
import functools

import jax
import jax.numpy as jnp
from jax import lax
from jax.experimental import pallas as pl
from jax.experimental.pallas import tpu as pltpu
from jax.experimental.pallas import tpu_sc as plsc

D = 128
CHUNK = 128
NC = 2
NS = 16
NW = NC * NS
FAST_FRAC = 0.627


def _agg_sc(node, adj_c, n_pad, tch, n0, n1):
    rpt = n_pad // NS
    nzc = rpt // CHUNK
    cpw_buf = -(-(max(n0, n1) + 16) // 8) * 8

    mesh = plsc.VectorSubcoreMesh(core_axis_name="c", subcore_axis_name="s")

    @functools.partial(
        pl.kernel,
        mesh=mesh,
        out_type=jax.ShapeDtypeStruct((NC * n_pad, D), jnp.float32),
        scratch_types=[
            pltpu.VMEM((cpw_buf, CHUNK), jnp.int32),
            pltpu.VMEM((cpw_buf, CHUNK), jnp.int32),
            pltpu.VMEM((CHUNK, D), jnp.float32),
            pltpu.VMEM_SHARED((n_pad, D), jnp.float32),
            pltpu.SemaphoreType.DMA,
        ],
    )
    def agg(node_hbm, adj_hbm, out_hbm, src_v, dst_v, rows_v, acc, sem):
        c = lax.axis_index("c")
        s = lax.axis_index("s")
        ncw = jnp.where(c == 0, n0, n1)
        o_w = jnp.where(c == 0, 0, NS * n0) + s * ncw
        my_cpw = jnp.maximum(0, jnp.minimum(ncw, tch - o_w))

        def window(plane_off):
            st = jnp.minimum(plane_off + o_w, 2 * tch - cpw_buf)
            st = pl.multiple_of((st // 8) * 8, 8)
            return st, plane_off + o_w - st

        s_start, s_base = window(0)
        d_start, d_base = window(tch)

        zero16 = jnp.zeros((16,), jnp.float32)

        def zbody(k, carry):
            i = k // (D // 16)
            j = k % (D // 16)
            rows_v[i, pl.ds(j * 16, 16)] = zero16
            return carry

        lax.fori_loop(0, CHUNK * (D // 16), zbody, 0)
        for k in range(nzc):
            pltpu.sync_copy(rows_v, acc.at[pl.ds(s * rpt + k * CHUNK, CHUNK)])
        plsc.subcore_barrier()

        pltpu.sync_copy(adj_hbm.at[pl.ds(s_start, cpw_buf)], src_v)
        pltpu.sync_copy(adj_hbm.at[pl.ds(d_start, cpw_buf)], dst_v)

        def body(j, carry):
            pltpu.async_copy(node_hbm.at[src_v.at[s_base + j]], rows_v, sem).wait()
            pltpu.sync_copy(rows_v, acc.at[dst_v.at[d_base + j]], add=True)
            return carry

        lax.fori_loop(0, my_cpw, body, 0)
        plsc.subcore_barrier()

        for k in range(nzc):
            row = s * rpt + k * CHUNK
            pltpu.sync_copy(acc.at[pl.ds(row, CHUNK)],
                            out_hbm.at[pl.ds(c * n_pad + row, CHUNK)])

    return agg(node, adj_c)


def _mlp_body(scale_ref, x_ref, p0_ref, p1_ref,
              w1_ref, b1_ref, g1_ref, be1_ref,
              w2_ref, b2_ref, g2_ref, be2_ref,
              w3_ref, b3_ref, gn_ref, bn_ref, out_ref):
    def ln_relu(h, g, b):
        mu = jnp.mean(h, axis=1, keepdims=True)
        xc = h - mu
        var = jnp.mean(xc * xc, axis=1, keepdims=True)
        return jnp.maximum(xc * lax.rsqrt(var + 1e-5) * g + b, 0.0)

    dn = (((1,), (1,)), ((), ()))
    h = scale_ref[0, 0] * x_ref[...] + p0_ref[...] + p1_ref[...]
    h = lax.dot_general(h, w1_ref[...], dn, preferred_element_type=jnp.float32)
    h = ln_relu(h + b1_ref[...], g1_ref[...], be1_ref[...])
    h = lax.dot_general(h, w2_ref[...], dn, preferred_element_type=jnp.float32)
    h = ln_relu(h + b2_ref[...], g2_ref[...], be2_ref[...])
    h = lax.dot_general(h, w3_ref[...], dn, preferred_element_type=jnp.float32)
    out_ref[...] = ln_relu(h + b3_ref[...], gn_ref[...], bn_ref[...])


def kernel(node, adj, batch_ptr, eps, W1, b1, g1, be1, W2, b2, g2, be2,
           W3, b3, gn, bn):
    n, d = node.shape
    e = adj.shape[1]
    assert d == D

    tch = -(-e // CHUNK)
    n0 = max(1, round(tch * FAST_FRAC / NS))
    n1 = max(1, -(-max(tch - NS * n0, 1) // NS))

    adjf = adj.astype(jnp.int32).reshape(2 * e // CHUNK, CHUNK)
    if e % CHUNK:
        pad = tch * CHUNK - e
        src = jnp.concatenate([adj[0].astype(jnp.int32), jnp.zeros((pad,), jnp.int32)])
        dst = jnp.concatenate([adj[1].astype(jnp.int32), jnp.full((pad,), n, jnp.int32)])
        adjf = jnp.concatenate([src, dst]).reshape(2 * tch, CHUNK)

    n_pad = -(-(n + 1) // (NS * CHUNK)) * NS * CHUNK
    parts = _agg_sc(node, adjf, n_pad, tch, n0, n1)

    scale = (1.0 + eps).astype(jnp.float32).reshape(1, 1)

    br = 512
    nb = -(-n // br)
    assert n_pad % br == 0
    po = n_pad // br
    full = lambda shp: pl.BlockSpec(shp, lambda i: (0, 0))
    row_blk = pl.BlockSpec((br, D), lambda i: (i, 0))
    p0_blk = pl.BlockSpec((br, D), lambda i: (i, 0))
    p1_blk = pl.BlockSpec((br, D), lambda i: (po + i, 0))
    vec = lambda: full((1, D))

    out = pl.pallas_call(
        _mlp_body,
        grid=(nb,),
        in_specs=[
            full((1, 1)),
            row_blk,
            p0_blk, p1_blk,
            full((D, D)), vec(), vec(), vec(),
            full((D, D)), vec(), vec(), vec(),
            full((D, D)), vec(), vec(), vec(),
        ],
        out_specs=row_blk,
        out_shape=jax.ShapeDtypeStruct((n, D), jnp.float32),
    )(
        scale, node, parts, parts,
        W1, b1.reshape(1, D), g1.reshape(1, D), be1.reshape(1, D),
        W2, b2.reshape(1, D), g2.reshape(1, D), be2.reshape(1, D),
        W3, b3.reshape(1, D), gn.reshape(1, D), bn.reshape(1, D),
    )
    return out


# speedup vs baseline: 8.1709x; 1.1740x over previous
import functools

import jax
import jax.numpy as jnp
from jax import lax
from jax.experimental import pallas as pl
from jax.experimental.pallas import tpu as pltpu
from jax.experimental.pallas import tpu_sc as plsc

D = 128
CHUNK = 128
NC = 2
NS = 16
NW = NC * NS
FAST_FRAC = 0.5


def _agg_sc(node, adj_c, n_pad, tch, n0, n1):
    rpt = n_pad // NS
    nzc = rpt // CHUNK
    cpw_buf = -(-(max(n0, n1) + 16) // 8) * 8

    mesh = plsc.VectorSubcoreMesh(core_axis_name="c", subcore_axis_name="s")

    @functools.partial(
        pl.kernel,
        mesh=mesh,
        out_type=jax.ShapeDtypeStruct((NC * n_pad, D), jnp.float32),
        scratch_types=[
            pltpu.VMEM((cpw_buf, CHUNK), jnp.int32),
            pltpu.VMEM((cpw_buf, CHUNK), jnp.int32),
            pltpu.VMEM((CHUNK, D), jnp.float32),
            pltpu.VMEM_SHARED((n_pad, D), jnp.float32),
            pltpu.SemaphoreType.DMA,
        ],
    )
    def agg(node_hbm, adj_hbm, out_hbm, src_v, dst_v, rows_v, acc, sem):
        c = lax.axis_index("c")
        s = lax.axis_index("s")
        ncw = jnp.where(c == 0, n0, n1)
        o_w = jnp.where(c == 0, 0, NS * n0) + s * ncw
        my_cpw = jnp.maximum(0, jnp.minimum(ncw, tch - o_w))

        def window(plane_off):
            st = jnp.minimum(plane_off + o_w, 2 * tch - cpw_buf)
            st = pl.multiple_of((st // 8) * 8, 8)
            return st, plane_off + o_w - st

        s_start, s_base = window(0)
        d_start, d_base = window(tch)

        zero16 = jnp.zeros((16,), jnp.float32)

        def zbody(k, carry):
            i = k // (D // 16)
            j = k % (D // 16)
            rows_v[i, pl.ds(j * 16, 16)] = zero16
            return carry

        lax.fori_loop(0, CHUNK * (D // 16), zbody, 0)
        for k in range(nzc):
            pltpu.sync_copy(rows_v, acc.at[pl.ds(s * rpt + k * CHUNK, CHUNK)])
        plsc.subcore_barrier()

        pltpu.sync_copy(adj_hbm.at[pl.ds(s_start, cpw_buf)], src_v)
        pltpu.sync_copy(adj_hbm.at[pl.ds(d_start, cpw_buf)], dst_v)

        def body(j, carry):
            pltpu.async_copy(node_hbm.at[src_v.at[s_base + j]], rows_v, sem).wait()
            pltpu.sync_copy(rows_v, acc.at[dst_v.at[d_base + j]], add=True)
            return carry

        lax.fori_loop(0, my_cpw, body, 0)
        plsc.subcore_barrier()

        for k in range(nzc):
            row = s * rpt + k * CHUNK
            pltpu.sync_copy(acc.at[pl.ds(row, CHUNK)],
                            out_hbm.at[pl.ds(c * n_pad + row, CHUNK)])

    return agg(node, adj_c)


def _mlp_body(scale_ref, x_ref, p0_ref, p1_ref,
              w1_ref, b1_ref, g1_ref, be1_ref,
              w2_ref, b2_ref, g2_ref, be2_ref,
              w3_ref, b3_ref, gn_ref, bn_ref, out_ref):
    def ln_relu(h, g, b):
        mu = jnp.mean(h, axis=1, keepdims=True)
        xc = h - mu
        var = jnp.mean(xc * xc, axis=1, keepdims=True)
        return jnp.maximum(xc * lax.rsqrt(var + 1e-5) * g + b, 0.0)

    dn = (((1,), (1,)), ((), ()))
    h = scale_ref[0, 0] * x_ref[...] + p0_ref[...] + p1_ref[...]
    h = lax.dot_general(h, w1_ref[...], dn, preferred_element_type=jnp.float32)
    h = ln_relu(h + b1_ref[...], g1_ref[...], be1_ref[...])
    h = lax.dot_general(h, w2_ref[...], dn, preferred_element_type=jnp.float32)
    h = ln_relu(h + b2_ref[...], g2_ref[...], be2_ref[...])
    h = lax.dot_general(h, w3_ref[...], dn, preferred_element_type=jnp.float32)
    out_ref[...] = ln_relu(h + b3_ref[...], gn_ref[...], bn_ref[...])


def kernel(node, adj, batch_ptr, eps, W1, b1, g1, be1, W2, b2, g2, be2,
           W3, b3, gn, bn):
    n, d = node.shape
    e = adj.shape[1]
    assert d == D

    tch = -(-e // CHUNK)
    n0 = max(1, round(tch * FAST_FRAC / NS))
    n1 = max(1, -(-max(tch - NS * n0, 1) // NS))

    adjf = adj.astype(jnp.int32).reshape(2 * e // CHUNK, CHUNK)
    if e % CHUNK:
        pad = tch * CHUNK - e
        src = jnp.concatenate([adj[0].astype(jnp.int32), jnp.zeros((pad,), jnp.int32)])
        dst = jnp.concatenate([adj[1].astype(jnp.int32), jnp.full((pad,), n, jnp.int32)])
        adjf = jnp.concatenate([src, dst]).reshape(2 * tch, CHUNK)

    n_pad = -(-(n + 1) // (NS * CHUNK)) * NS * CHUNK
    parts = _agg_sc(node, adjf, n_pad, tch, n0, n1)

    scale = (1.0 + eps).astype(jnp.float32).reshape(1, 1)

    br = 512
    nb = -(-n // br)
    assert n_pad % br == 0
    po = n_pad // br
    full = lambda shp: pl.BlockSpec(shp, lambda i: (0, 0))
    row_blk = pl.BlockSpec((br, D), lambda i: (i, 0))
    p0_blk = pl.BlockSpec((br, D), lambda i: (i, 0))
    p1_blk = pl.BlockSpec((br, D), lambda i: (po + i, 0))
    vec = lambda: full((1, D))

    out = pl.pallas_call(
        _mlp_body,
        grid=(nb,),
        in_specs=[
            full((1, 1)),
            row_blk,
            p0_blk, p1_blk,
            full((D, D)), vec(), vec(), vec(),
            full((D, D)), vec(), vec(), vec(),
            full((D, D)), vec(), vec(), vec(),
        ],
        out_specs=row_blk,
        out_shape=jax.ShapeDtypeStruct((n, D), jnp.float32),
    )(
        scale, node, parts, parts,
        W1, b1.reshape(1, D), g1.reshape(1, D), be1.reshape(1, D),
        W2, b2.reshape(1, D), g2.reshape(1, D), be2.reshape(1, D),
        W3, b3.reshape(1, D), gn.reshape(1, D), bn.reshape(1, D),
    )
    return out


# speedup vs baseline: 11.0987x; 1.3583x over previous
import functools

import jax
import jax.numpy as jnp
from jax import lax
from jax.experimental import pallas as pl
from jax.experimental.pallas import tpu as pltpu
from jax.experimental.pallas import tpu_sc as plsc

D = 128
CHUNK = 128
NC = 2
NS = 16
NW = NC * NS
FAST_FRAC = 0.5


def _agg_sc(node, pk_c, n_pad, tch, n0, n1):
    rpt = n_pad // NS
    nzc = rpt // CHUNK
    cpw_buf = -(-max(n0, n1) // 8) * 8 + 8
    ta = pk_c.shape[0]
    K = 2

    mesh = plsc.VectorSubcoreMesh(core_axis_name="c", subcore_axis_name="s")

    @functools.partial(
        pl.kernel,
        mesh=mesh,
        out_type=jax.ShapeDtypeStruct((NC * n_pad, D), jnp.float32),
        scratch_types=[
            pltpu.VMEM((cpw_buf, CHUNK), jnp.int32),
            [pltpu.VMEM((K, CHUNK), jnp.int32)] * 2,
            [pltpu.VMEM((CHUNK, D), jnp.float32)] * K,
            pltpu.VMEM_SHARED((n_pad, D), jnp.float32),
            [pltpu.SemaphoreType.DMA] * K,
        ],
    )
    def agg(node_hbm, pk_hbm, out_hbm, pk_v, sd_v, rows_v, acc, gsems):
        c = lax.axis_index("c")
        s = lax.axis_index("s")
        sidx, didx = sd_v
        ncw = jnp.where(c == 0, n0, n1)
        o_w = jnp.where(c == 0, 0, NS * n0) + s * ncw
        my_cpw = jnp.maximum(0, jnp.minimum(ncw, tch - o_w))
        start = jnp.maximum(0, jnp.minimum(o_w, ta - cpw_buf))
        start = pl.multiple_of((start // 8) * 8, 8)
        base = o_w - start

        zero16 = jnp.zeros((16,), jnp.float32)

        def zbody(k, carry):
            i = k // (D // 16)
            j = k % (D // 16)
            rows_v[0][i, pl.ds(j * 16, 16)] = zero16
            return carry

        lax.fori_loop(0, CHUNK * (D // 16), zbody, 0)
        for k in range(nzc):
            pltpu.sync_copy(rows_v[0], acc.at[pl.ds(s * rpt + k * CHUNK, CHUNK)])
        plsc.subcore_barrier()

        pltpu.sync_copy(pk_hbm.at[pl.ds(start, cpw_buf)], pk_v)

        def unpack(j, b):
            for i in range(CHUNK // 16):
                w = pk_v[base + j, pl.ds(i * 16, 16)]
                sidx[b, pl.ds(i * 16, 16)] = lax.bitwise_and(w, 0xFFFF)
                didx[b, pl.ds(i * 16, 16)] = lax.shift_right_logical(w, 16)

        def fire(j, b):
            unpack(j, b)
            pltpu.async_copy(node_hbm.at[sidx.at[b]], rows_v[b], gsems[b])

        for b in range(K):
            @pl.when(b < my_cpw)
            def _():
                fire(b, b)

        def body(t, carry):
            j0 = t * K
            for b in range(K):
                j = j0 + b
                pltpu.make_async_copy(
                    node_hbm.at[sidx.at[b]], rows_v[b], gsems[b]).wait()
                pltpu.sync_copy(rows_v[b], acc.at[didx.at[b]], add=True)

                @pl.when(j + K < my_cpw)
                def _():
                    fire(j + K, b)
            return carry

        lax.fori_loop(0, my_cpw // K, body, 0)
        plsc.subcore_barrier()

        for k in range(nzc):
            row = s * rpt + k * CHUNK
            pltpu.sync_copy(acc.at[pl.ds(row, CHUNK)],
                            out_hbm.at[pl.ds(c * n_pad + row, CHUNK)])

    return agg(node, pk_c)


def _mlp_body(scale_ref, x_ref, p0_ref, p1_ref,
              w1_ref, b1_ref, g1_ref, be1_ref,
              w2_ref, b2_ref, g2_ref, be2_ref,
              w3_ref, b3_ref, gn_ref, bn_ref, out_ref):
    def ln_relu(h, g, b):
        mu = jnp.mean(h, axis=1, keepdims=True)
        xc = h - mu
        var = jnp.mean(xc * xc, axis=1, keepdims=True)
        return jnp.maximum(xc * lax.rsqrt(var + 1e-5) * g + b, 0.0)

    dn = (((1,), (1,)), ((), ()))
    h = scale_ref[0, 0] * x_ref[...] + p0_ref[...] + p1_ref[...]
    h = lax.dot_general(h, w1_ref[...], dn, preferred_element_type=jnp.float32)
    h = ln_relu(h + b1_ref[...], g1_ref[...], be1_ref[...])
    h = lax.dot_general(h, w2_ref[...], dn, preferred_element_type=jnp.float32)
    h = ln_relu(h + b2_ref[...], g2_ref[...], be2_ref[...])
    h = lax.dot_general(h, w3_ref[...], dn, preferred_element_type=jnp.float32)
    out_ref[...] = ln_relu(h + b3_ref[...], gn_ref[...], bn_ref[...])


def kernel(node, adj, batch_ptr, eps, W1, b1, g1, be1, W2, b2, g2, be2,
           W3, b3, gn, bn):
    n, d = node.shape
    e = adj.shape[1]
    assert d == D

    tch = -(-e // CHUNK)
    assert tch % 2 == 0
    n0 = 2 * max(1, round(tch * FAST_FRAC / (2 * NS)))
    n1 = 2 * max(1, -(-max(tch - NS * n0, 1) // (2 * NS)))

    src = adj[0].astype(jnp.int32)
    dst = adj[1].astype(jnp.int32)
    if e % CHUNK:
        pad = tch * CHUNK - e
        src = jnp.concatenate([src, jnp.zeros((pad,), jnp.int32)])
        dst = jnp.concatenate([dst, jnp.full((pad,), n, jnp.int32)])
    ta = ((tch + 7) // 8) * 8 + 8
    pk = jnp.concatenate(
        [src | (dst << 16), jnp.zeros((ta * CHUNK - tch * CHUNK,), jnp.int32)])
    pk_c = pk.reshape(ta, CHUNK)

    n_pad = -(-(n + 1) // (NS * CHUNK)) * NS * CHUNK
    parts = _agg_sc(node, pk_c, n_pad, tch, n0, n1)

    scale = (1.0 + eps).astype(jnp.float32).reshape(1, 1)

    br = 512
    nb = -(-n // br)
    assert n_pad % br == 0
    po = n_pad // br
    full = lambda shp: pl.BlockSpec(shp, lambda i: (0, 0))
    row_blk = pl.BlockSpec((br, D), lambda i: (i, 0))
    p0_blk = pl.BlockSpec((br, D), lambda i: (i, 0))
    p1_blk = pl.BlockSpec((br, D), lambda i: (po + i, 0))
    vec = lambda: full((1, D))

    out = pl.pallas_call(
        _mlp_body,
        grid=(nb,),
        in_specs=[
            full((1, 1)),
            row_blk,
            p0_blk, p1_blk,
            full((D, D)), vec(), vec(), vec(),
            full((D, D)), vec(), vec(), vec(),
            full((D, D)), vec(), vec(), vec(),
        ],
        out_specs=row_blk,
        out_shape=jax.ShapeDtypeStruct((n, D), jnp.float32),
    )(
        scale, node, parts, parts,
        W1, b1.reshape(1, D), g1.reshape(1, D), be1.reshape(1, D),
        W2, b2.reshape(1, D), g2.reshape(1, D), be2.reshape(1, D),
        W3, b3.reshape(1, D), gn.reshape(1, D), bn.reshape(1, D),
    )
    return out


# speedup vs baseline: 11.1015x; 1.0003x over previous
import functools

import jax
import jax.numpy as jnp
from jax import lax
from jax.experimental import pallas as pl
from jax.experimental.pallas import tpu as pltpu
from jax.experimental.pallas import tpu_sc as plsc

D = 128
CHUNK = 128
NC = 2
NS = 16
NW = NC * NS
K = 2
QI = 4


def _agg_sc(node, src_e, dst_e, n_pad, tch, n0, n1):
    rpt = n_pad // NS
    nzc = rpt // CHUNK

    mesh = plsc.VectorSubcoreMesh(core_axis_name="c", subcore_axis_name="s")

    @functools.partial(
        pl.kernel,
        mesh=mesh,
        out_type=jax.ShapeDtypeStruct((NC * n_pad, D), jnp.float32),
        scratch_types=[
            [pltpu.VMEM((QI, CHUNK), jnp.int32)] * 2,
            [pltpu.VMEM((CHUNK, D), jnp.float32)] * K,
            pltpu.VMEM_SHARED((n_pad, D), jnp.float32),
            [pltpu.SemaphoreType.DMA] * QI,
            [pltpu.SemaphoreType.DMA] * QI,
            [pltpu.SemaphoreType.DMA] * K,
        ],
    )
    def agg(node_hbm, src_hbm, dst_hbm, out_hbm, sd_v, rows_v, acc,
            isems, dsems, gsems):
        c = lax.axis_index("c")
        s = lax.axis_index("s")
        sidx, didx = sd_v
        ncw = jnp.where(c == 0, n0, n1)
        o_w = jnp.where(c == 0, 0, NS * n0) + s * ncw
        my_cpw = jnp.maximum(0, jnp.minimum(ncw, tch - o_w))

        zero16 = jnp.zeros((16,), jnp.float32)

        def zbody(k, carry):
            i = k // (D // 16)
            j = k % (D // 16)
            rows_v[0][i, pl.ds(j * 16, 16)] = zero16
            return carry

        lax.fori_loop(0, CHUNK * (D // 16), zbody, 0)
        for k in range(nzc):
            pltpu.sync_copy(rows_v[0], acc.at[pl.ds(s * rpt + k * CHUNK, CHUNK)])
        plsc.subcore_barrier()

        def fire_idx(j, q):
            e0 = (o_w + j) * CHUNK
            pltpu.async_copy(src_hbm.at[pl.ds(e0, CHUNK)], sidx.at[q], isems[q])
            pltpu.async_copy(dst_hbm.at[pl.ds(e0, CHUNK)], didx.at[q], dsems[q])

        def wait_idx(sems, q):
            pltpu.make_async_copy(
                src_hbm.at[pl.ds(0, CHUNK)], sidx.at[q], sems[q]).wait()

        def fire_gather(q, b):
            pltpu.async_copy(node_hbm.at[sidx.at[q]], rows_v[b], gsems[b])

        for q in range(QI):
            @pl.when(q < my_cpw)
            def _():
                fire_idx(q, q)
        for b in range(K):
            @pl.when(b < my_cpw)
            def _():
                wait_idx(isems, b)
                fire_gather(b, b)

        def body(t, carry):
            j0 = t * QI
            for u in range(QI):
                j = j0 + u
                b = u % K
                pltpu.make_async_copy(
                    node_hbm.at[sidx.at[u]], rows_v[b], gsems[b]).wait()
                wait_idx(dsems, u)
                pltpu.sync_copy(rows_v[b], acc.at[didx.at[u]], add=True)

                @pl.when(j + QI < my_cpw)
                def _():
                    fire_idx(j + QI, u)

                @pl.when(j + K < my_cpw)
                def _():
                    wait_idx(isems, (u + K) % QI)
                    fire_gather((u + K) % QI, b)
            return carry

        lax.fori_loop(0, my_cpw // QI, body, 0)
        plsc.subcore_barrier()

        for k in range(nzc):
            row = s * rpt + k * CHUNK
            pltpu.sync_copy(acc.at[pl.ds(row, CHUNK)],
                            out_hbm.at[pl.ds(c * n_pad + row, CHUNK)])

    return agg(node, src_e, dst_e)


def _mlp_body(scale_ref, x_ref, p0_ref, p1_ref,
              w1_ref, b1_ref, g1_ref, be1_ref,
              w2_ref, b2_ref, g2_ref, be2_ref,
              w3_ref, b3_ref, gn_ref, bn_ref, out_ref):
    def ln_relu(h, g, b):
        mu = jnp.mean(h, axis=1, keepdims=True)
        xc = h - mu
        var = jnp.mean(xc * xc, axis=1, keepdims=True)
        return jnp.maximum(xc * lax.rsqrt(var + 1e-5) * g + b, 0.0)

    dn = (((1,), (1,)), ((), ()))
    h = scale_ref[0, 0] * x_ref[...] + p0_ref[...] + p1_ref[...]
    h = lax.dot_general(h, w1_ref[...], dn, preferred_element_type=jnp.float32)
    h = ln_relu(h + b1_ref[...], g1_ref[...], be1_ref[...])
    h = lax.dot_general(h, w2_ref[...], dn, preferred_element_type=jnp.float32)
    h = ln_relu(h + b2_ref[...], g2_ref[...], be2_ref[...])
    h = lax.dot_general(h, w3_ref[...], dn, preferred_element_type=jnp.float32)
    out_ref[...] = ln_relu(h + b3_ref[...], gn_ref[...], bn_ref[...])


def kernel(node, adj, batch_ptr, eps, W1, b1, g1, be1, W2, b2, g2, be2,
           W3, b3, gn, bn):
    n, d = node.shape
    e = adj.shape[1]
    assert d == D

    tch = -(-e // CHUNK)
    assert tch % QI == 0
    n0 = QI * max(1, -(-tch // (2 * NS * QI)))
    n1 = n0

    src = adj[0].astype(jnp.int32)
    dst = adj[1].astype(jnp.int32)
    if e % CHUNK:
        pad = tch * CHUNK - e
        src = jnp.concatenate([src, jnp.zeros((pad,), jnp.int32)])
        dst = jnp.concatenate([dst, jnp.full((pad,), n, jnp.int32)])

    n_pad = -(-(n + 1) // (NS * CHUNK)) * NS * CHUNK
    parts = _agg_sc(node, src, dst, n_pad, tch, n0, n1)

    scale = (1.0 + eps).astype(jnp.float32).reshape(1, 1)

    br = 512
    nb = -(-n // br)
    assert n_pad % br == 0
    po = n_pad // br
    full = lambda shp: pl.BlockSpec(shp, lambda i: (0, 0))
    row_blk = pl.BlockSpec((br, D), lambda i: (i, 0))
    p0_blk = pl.BlockSpec((br, D), lambda i: (i, 0))
    p1_blk = pl.BlockSpec((br, D), lambda i: (po + i, 0))
    vec = lambda: full((1, D))

    out = pl.pallas_call(
        _mlp_body,
        grid=(nb,),
        in_specs=[
            full((1, 1)),
            row_blk,
            p0_blk, p1_blk,
            full((D, D)), vec(), vec(), vec(),
            full((D, D)), vec(), vec(), vec(),
            full((D, D)), vec(), vec(), vec(),
        ],
        out_specs=row_blk,
        out_shape=jax.ShapeDtypeStruct((n, D), jnp.float32),
    )(
        scale, node, parts, parts,
        W1, b1.reshape(1, D), g1.reshape(1, D), be1.reshape(1, D),
        W2, b2.reshape(1, D), g2.reshape(1, D), be2.reshape(1, D),
        W3, b3.reshape(1, D), gn.reshape(1, D), bn.reshape(1, D),
    )
    return out


# speedup vs baseline: 12.0004x; 1.0810x over previous
import functools

import jax
import jax.numpy as jnp
from jax import lax
from jax.experimental import pallas as pl
from jax.experimental.pallas import tpu as pltpu
from jax.experimental.pallas import tpu_sc as plsc

D = 128
CHUNK = 128
NC = 2
NS = 16
NW = NC * NS
K = 2
QI = 4


def _agg_sc(node, adj_e, n_pad, tch, n0, n1):
    rpt = n_pad // NS
    nzc = rpt // CHUNK

    mesh = plsc.VectorSubcoreMesh(core_axis_name="c", subcore_axis_name="s")

    @functools.partial(
        pl.kernel,
        mesh=mesh,
        out_type=jax.ShapeDtypeStruct((NC * n_pad, D), jnp.float32),
        scratch_types=[
            [pltpu.VMEM((QI, CHUNK), jnp.int32)] * 2,
            [pltpu.VMEM((CHUNK, D), jnp.float32)] * K,
            pltpu.VMEM_SHARED((n_pad, D), jnp.float32),
            [pltpu.SemaphoreType.DMA] * QI,
            [pltpu.SemaphoreType.DMA] * QI,
            [pltpu.SemaphoreType.DMA] * K,
        ],
    )
    def agg(node_hbm, adj_hbm, out_hbm, sd_v, rows_v, acc,
            isems, dsems, gsems):
        c = lax.axis_index("c")
        s = lax.axis_index("s")
        sidx, didx = sd_v
        ncw = jnp.where(c == 0, n0, n1)
        o_w = jnp.where(c == 0, 0, NS * n0) + s * ncw
        my_cpw = jnp.maximum(0, jnp.minimum(ncw, tch - o_w))

        zero16 = jnp.zeros((16,), jnp.float32)

        def zbody(k, carry):
            i = k // (D // 16)
            j = k % (D // 16)
            rows_v[0][i, pl.ds(j * 16, 16)] = zero16
            return carry

        lax.fori_loop(0, CHUNK * (D // 16), zbody, 0)
        for k in range(nzc):
            pltpu.sync_copy(rows_v[0], acc.at[pl.ds(s * rpt + k * CHUNK, CHUNK)])
        plsc.subcore_barrier()

        def fire_idx(j, q):
            e0 = (o_w + j) * CHUNK
            pltpu.async_copy(adj_hbm.at[0, pl.ds(e0, CHUNK)], sidx.at[q], isems[q])
            pltpu.async_copy(adj_hbm.at[1, pl.ds(e0, CHUNK)], didx.at[q], dsems[q])

        def wait_idx(sems, q):
            pltpu.make_async_copy(
                adj_hbm.at[0, pl.ds(0, CHUNK)], sidx.at[q], sems[q]).wait()

        def fire_gather(q, b):
            pltpu.async_copy(node_hbm.at[sidx.at[q]], rows_v[b], gsems[b])

        for q in range(QI):
            @pl.when(q < my_cpw)
            def _():
                fire_idx(q, q)
        for b in range(K):
            @pl.when(b < my_cpw)
            def _():
                wait_idx(isems, b)
                fire_gather(b, b)

        def body(t, carry):
            j0 = t * QI
            for u in range(QI):
                j = j0 + u
                b = u % K
                pltpu.make_async_copy(
                    node_hbm.at[sidx.at[u]], rows_v[b], gsems[b]).wait()
                wait_idx(dsems, u)
                pltpu.sync_copy(rows_v[b], acc.at[didx.at[u]], add=True)

                @pl.when(j + QI < my_cpw)
                def _():
                    fire_idx(j + QI, u)

                @pl.when(j + K < my_cpw)
                def _():
                    wait_idx(isems, (u + K) % QI)
                    fire_gather((u + K) % QI, b)
            return carry

        lax.fori_loop(0, my_cpw // QI, body, 0)
        plsc.subcore_barrier()

        for k in range(nzc):
            row = s * rpt + k * CHUNK
            pltpu.sync_copy(acc.at[pl.ds(row, CHUNK)],
                            out_hbm.at[pl.ds(c * n_pad + row, CHUNK)])

    return agg(node, adj_e)


def _mlp_body(scale_ref, x_ref, p0_ref, p1_ref,
              w1_ref, b1_ref, g1_ref, be1_ref,
              w2_ref, b2_ref, g2_ref, be2_ref,
              w3_ref, b3_ref, gn_ref, bn_ref, out_ref):
    def ln_relu(h, g, b):
        mu = jnp.mean(h, axis=1, keepdims=True)
        xc = h - mu
        var = jnp.mean(xc * xc, axis=1, keepdims=True)
        return jnp.maximum(xc * lax.rsqrt(var + 1e-5) * g + b, 0.0)

    dn = (((1,), (1,)), ((), ()))
    h = scale_ref[0, 0] * x_ref[...] + p0_ref[...] + p1_ref[...]
    h = lax.dot_general(h, w1_ref[...], dn, preferred_element_type=jnp.float32)
    h = ln_relu(h + b1_ref[...], g1_ref[...], be1_ref[...])
    h = lax.dot_general(h, w2_ref[...], dn, preferred_element_type=jnp.float32)
    h = ln_relu(h + b2_ref[...], g2_ref[...], be2_ref[...])
    h = lax.dot_general(h, w3_ref[...], dn, preferred_element_type=jnp.float32)
    out_ref[...] = ln_relu(h + b3_ref[...], gn_ref[...], bn_ref[...])


def kernel(node, adj, batch_ptr, eps, W1, b1, g1, be1, W2, b2, g2, be2,
           W3, b3, gn, bn):
    n, d = node.shape
    e = adj.shape[1]
    assert d == D

    tch = -(-e // CHUNK)
    assert tch % QI == 0
    n0 = QI * max(1, -(-tch // (2 * NS * QI)))
    n1 = n0

    adj_e = adj.astype(jnp.int32)
    if e % CHUNK:
        pad = tch * CHUNK - e
        adj_e = jnp.concatenate(
            [adj_e,
             jnp.stack([jnp.zeros((pad,), jnp.int32),
                        jnp.full((pad,), n, jnp.int32)])], axis=1)

    n_pad = -(-(n + 1) // (NS * CHUNK)) * NS * CHUNK
    parts = _agg_sc(node, adj_e, n_pad, tch, n0, n1)

    scale = (1.0 + eps).astype(jnp.float32).reshape(1, 1)

    br = 512
    nb = -(-n // br)
    assert n_pad % br == 0
    po = n_pad // br
    full = lambda shp: pl.BlockSpec(shp, lambda i: (0, 0))
    row_blk = pl.BlockSpec((br, D), lambda i: (i, 0))
    p0_blk = pl.BlockSpec((br, D), lambda i: (i, 0))
    p1_blk = pl.BlockSpec((br, D), lambda i: (po + i, 0))
    vec = lambda: full((1, D))

    out = pl.pallas_call(
        _mlp_body,
        grid=(nb,),
        in_specs=[
            full((1, 1)),
            row_blk,
            p0_blk, p1_blk,
            full((D, D)), vec(), vec(), vec(),
            full((D, D)), vec(), vec(), vec(),
            full((D, D)), vec(), vec(), vec(),
        ],
        out_specs=row_blk,
        out_shape=jax.ShapeDtypeStruct((n, D), jnp.float32),
    )(
        scale, node, parts, parts,
        W1, b1.reshape(1, D), g1.reshape(1, D), be1.reshape(1, D),
        W2, b2.reshape(1, D), g2.reshape(1, D), be2.reshape(1, D),
        W3, b3.reshape(1, D), gn.reshape(1, D), bn.reshape(1, D),
    )
    return out
